# baseline (speedup 1.0000x reference)
import functools
import jax
import jax.numpy as jnp
from jax import lax
from jax.experimental import pallas as pl
from jax.experimental.pallas import tpu as pltpu
from jax.experimental.pallas import tpu_sc as plsc

B = 16384
D = 64
EU = 1000000
EM = 100000
NC = 2
NS = 16
NW = NC * NS
RPW = B // NW
L = 16
NCHUNK = RPW // L


def _gather_table(tbl, ids_v, rows_g, stage_v, out, wid, sem):
    def chunk(i, _):
        base = i * L
        idv = ids_v[pl.ds(base, L)]
        tid = idv >> 3
        sub = idv & 7
        copies = [
            pltpu.async_copy(tbl.at[pl.ds(tid[j] * 8, 8), :], rows_g.at[j], sem)
            for j in range(L)
        ]
        for c in copies:
            c.wait()
        for j in range(L):
            sj = sub[j]
            for k in range(0, D, L):
                stage_v[j, pl.ds(k, L)] = rows_g[j, sj, pl.ds(k, L)]
        pltpu.sync_copy(stage_v, out.at[pl.ds(wid * RPW + base, L), :])
        return ()

    lax.fori_loop(0, NCHUNK, chunk, (), unroll=False)


def _gather_body(ut, mt, uids_hbm, mids_hbm, ue, me,
                 uids_v, mids_v, rows_g, stage_v, sem):
    wid = lax.axis_index("s") * NC + lax.axis_index("c")
    rbase = wid * RPW
    pltpu.sync_copy(uids_hbm.at[pl.ds(rbase, RPW)], uids_v)
    pltpu.sync_copy(mids_hbm.at[pl.ds(rbase, RPW)], mids_v)
    _gather_table(ut, uids_v, rows_g, stage_v, ue, wid, sem)
    _gather_table(mt, mids_v, rows_g, stage_v, me, wid, sem)


@functools.lru_cache(maxsize=1)
def _make_gather_call():
    return functools.partial(
        pl.kernel,
        out_type=(
            jax.ShapeDtypeStruct((B, D), jnp.float32),
            jax.ShapeDtypeStruct((B, D), jnp.float32),
        ),
        mesh=plsc.VectorSubcoreMesh(core_axis_name="c", subcore_axis_name="s"),
        scratch_types=[
            pltpu.VMEM((RPW,), jnp.int32),
            pltpu.VMEM((RPW,), jnp.int32),
            pltpu.VMEM((L, 8, D), jnp.float32),
            pltpu.VMEM((L, D), jnp.float32),
            pltpu.SemaphoreType.DMA,
        ],
    )(_gather_body)


BLK = 2048


def _dense_body(ue, me, uf, mf, uW1, ub1, uW2, ub2, mW1, mb1, mW2, mb2,
                dW1, db1, dW2, db2, dW3, db3, out):
    f32 = jnp.float32
    ufe = jnp.maximum(jnp.dot(uf[...], uW1[...], preferred_element_type=f32)
                      + ub1[...], 0.0)
    ufe = jnp.dot(ufe, uW2[...], preferred_element_type=f32) + ub2[...]
    mfe = jnp.maximum(jnp.dot(mf[...], mW1[...], preferred_element_type=f32)
                      + mb1[...], 0.0)
    mfe = jnp.dot(mfe, mW2[...], preferred_element_type=f32) + mb2[...]
    h = (jnp.dot(ue[...], dW1[0:64, :], preferred_element_type=f32)
         + jnp.dot(me[...], dW1[64:128, :], preferred_element_type=f32)
         + jnp.dot(ufe, dW1[128:160, :], preferred_element_type=f32)
         + jnp.dot(mfe, dW1[160:192, :], preferred_element_type=f32)
         + db1[...])
    h = jnp.maximum(h, 0.0)
    h = jnp.maximum(jnp.dot(h, dW2[...], preferred_element_type=f32)
                    + db2[...], 0.0)
    out[...] = jnp.dot(h, dW3[...], preferred_element_type=f32) + db3[...]


def _dense_call(ue, me, uf, mf, uW1, ub1, uW2, ub2, mW1, mb1, mW2, mb2,
                dW1, db1, dW2, db2, dW3, db3, interpret=False):
    row_spec = pl.BlockSpec((BLK, D), lambda i: (i, 0))
    full = lambda a: pl.BlockSpec(a.shape, lambda i: tuple(0 for _ in a.shape))
    args = (ue, me, uf, mf, uW1, ub1, uW2, ub2, mW1, mb1, mW2, mb2,
            dW1, db1, dW2, db2, dW3, db3)
    in_specs = [row_spec, row_spec, row_spec, row_spec] + [full(a) for a in args[4:]]
    return pl.pallas_call(
        _dense_body,
        grid=(B // BLK,),
        in_specs=in_specs,
        out_specs=pl.BlockSpec((BLK, 1), lambda i: (i, 0)),
        out_shape=jax.ShapeDtypeStruct((B, 1), jnp.float32),
        interpret=interpret,
    )(*args)


def kernel(user_ids, movie_ids, user_features, movie_features, user_table,
           movie_table, uW1, ub1, uW2, ub2, mW1, mb1, mW2, mb2,
           dW1, db1, dW2, db2, dW3, db3):
    uids = user_ids.astype(jnp.int32)
    mids = movie_ids.astype(jnp.int32)
    ue, me = _make_gather_call()(user_table, movie_table, uids, mids)
    out = _dense_call(
        ue, me, user_features, movie_features,
        uW1, ub1.reshape(1, -1), uW2, ub2.reshape(1, -1),
        mW1, mb1.reshape(1, -1), mW2, mb2.reshape(1, -1),
        dW1, db1.reshape(1, -1), dW2, db2.reshape(1, -1),
        dW3, db3.reshape(1, -1))
    return out[:, 0]


# speedup vs baseline: 1.0497x; 1.0497x over previous
import functools
import jax
import jax.numpy as jnp
from jax import lax
from jax.experimental import pallas as pl
from jax.experimental.pallas import tpu as pltpu
from jax.experimental.pallas import tpu_sc as plsc

B = 16384
D = 64
NC = 2
NS = 16
NW = NC * NS
RPW = B // NW
L = 16
NCHUNK = RPW // L


def _gather_body(tbl, ids_hbm, out, ids_v, rows_g, stage_v, sem):
    wid = lax.axis_index("s") * NC + lax.axis_index("c")
    rbase = wid * RPW
    pltpu.sync_copy(ids_hbm.at[pl.ds(rbase, RPW)], ids_v)

    def chunk(i, _):
        base = i * L
        idv = ids_v[pl.ds(base, L)]
        tid = idv >> 3
        sub = idv & 7
        copies = [
            pltpu.async_copy(tbl.at[pl.ds(tid[j] * 8, 8), :], rows_g.at[j], sem)
            for j in range(L)
        ]
        for c in copies:
            c.wait()
        for j in range(L):
            sj = sub[j]
            for k in range(0, D, L):
                stage_v[j, pl.ds(k, L)] = rows_g[j, sj, pl.ds(k, L)]
        pltpu.sync_copy(stage_v, out.at[pl.ds(rbase + base, L), :])
        return ()

    lax.fori_loop(0, NCHUNK, chunk, (), unroll=False)


@functools.lru_cache(maxsize=1)
def _make_gather_call():
    return functools.partial(
        pl.kernel,
        out_type=jax.ShapeDtypeStruct((B, D), jnp.float32),
        mesh=plsc.VectorSubcoreMesh(core_axis_name="c", subcore_axis_name="s"),
        scratch_types=[
            pltpu.VMEM((RPW,), jnp.int32),
            pltpu.VMEM((L, 8, D), jnp.float32),
            pltpu.VMEM((L, D), jnp.float32),
            pltpu.SemaphoreType.DMA,
        ],
    )(_gather_body)


BLK = 2048


def _feat_body(uf, mf, uW1, ub1, uW2, ub2, mW1, mb1, mW2, mb2,
               dW1u, dW1m, db1, out):
    f32 = jnp.float32
    ufe = jnp.maximum(jnp.dot(uf[...], uW1[...], preferred_element_type=f32)
                      + ub1[...], 0.0)
    ufe = jnp.dot(ufe, uW2[...], preferred_element_type=f32) + ub2[...]
    mfe = jnp.maximum(jnp.dot(mf[...], mW1[...], preferred_element_type=f32)
                      + mb1[...], 0.0)
    mfe = jnp.dot(mfe, mW2[...], preferred_element_type=f32) + mb2[...]
    out[...] = (jnp.dot(ufe, dW1u[...], preferred_element_type=f32)
                + jnp.dot(mfe, dW1m[...], preferred_element_type=f32)
                + db1[...])


def _feat_call(uf, mf, uW1, ub1, uW2, ub2, mW1, mb1, mW2, mb2, dW1u, dW1m, db1):
    row_spec = pl.BlockSpec((BLK, D), lambda i: (i, 0))
    full = lambda a: pl.BlockSpec(a.shape, lambda i: tuple(0 for _ in a.shape))
    args = (uf, mf, uW1, ub1, uW2, ub2, mW1, mb1, mW2, mb2, dW1u, dW1m, db1)
    in_specs = [row_spec, row_spec] + [full(a) for a in args[2:]]
    return pl.pallas_call(
        _feat_body,
        grid=(B // BLK,),
        in_specs=in_specs,
        out_specs=pl.BlockSpec((BLK, 128), lambda i: (i, 0)),
        out_shape=jax.ShapeDtypeStruct((B, 128), jnp.float32),
    )(*args)


def _head_body(ue, me, hf, dW1a, dW1b, dW2, db2, dW3, db3, out):
    f32 = jnp.float32
    h = (jnp.dot(ue[...], dW1a[...], preferred_element_type=f32)
         + jnp.dot(me[...], dW1b[...], preferred_element_type=f32)
         + hf[...])
    h = jnp.maximum(h, 0.0)
    h = jnp.maximum(jnp.dot(h, dW2[...], preferred_element_type=f32)
                    + db2[...], 0.0)
    out[...] = jnp.dot(h, dW3[...], preferred_element_type=f32) + db3[...]


def _head_call(ue, me, hf, dW1a, dW1b, dW2, db2, dW3, db3):
    row_spec = pl.BlockSpec((BLK, D), lambda i: (i, 0))
    full = lambda a: pl.BlockSpec(a.shape, lambda i: tuple(0 for _ in a.shape))
    args = (ue, me, hf, dW1a, dW1b, dW2, db2, dW3, db3)
    in_specs = [row_spec, row_spec, pl.BlockSpec((BLK, 128), lambda i: (i, 0))]
    in_specs += [full(a) for a in args[3:]]
    return pl.pallas_call(
        _head_body,
        grid=(B // BLK,),
        in_specs=in_specs,
        out_specs=pl.BlockSpec((BLK, 1), lambda i: (i, 0)),
        out_shape=jax.ShapeDtypeStruct((B, 1), jnp.float32),
    )(*args)


def kernel(user_ids, movie_ids, user_features, movie_features, user_table,
           movie_table, uW1, ub1, uW2, ub2, mW1, mb1, mW2, mb2,
           dW1, db1, dW2, db2, dW3, db3):
    uids = user_ids.astype(jnp.int32)
    mids = movie_ids.astype(jnp.int32)
    gather = _make_gather_call()
    me = gather(movie_table, mids)
    ue = gather(user_table, uids)
    hf = _feat_call(
        user_features, movie_features,
        uW1, ub1.reshape(1, -1), uW2, ub2.reshape(1, -1),
        mW1, mb1.reshape(1, -1), mW2, mb2.reshape(1, -1),
        dW1[128:160], dW1[160:192], db1.reshape(1, -1))
    out = _head_call(ue, me, hf, dW1[0:64], dW1[64:128],
                     dW2, db2.reshape(1, -1), dW3, db3.reshape(1, 1))
    return out[:, 0]
